# Initial kernel scaffold; baseline (speedup 1.0000x reference)
import jax
import jax.numpy as jnp
from jax.experimental import pallas as pl


def kernel(x, adj, W1, b1, W2, b2):
    raise NotImplementedError("write your pallas kernel here")


# speedup vs baseline: 1.0032x; 1.0032x over previous
import jax
import jax.numpy as jnp
from jax.experimental import pallas as pl

_F = 128
_BM = 400


def _xw_kernel(x_ref, w_ref, o_ref):
    o_ref[...] = jnp.dot(
        x_ref[...], w_ref[...], preferred_element_type=jnp.float32
    )


def _layer1_kernel(adj_ref, s1_ref, b1_ref, w2_ref, o_ref):
    t = jnp.dot(adj_ref[...], s1_ref[...], preferred_element_type=jnp.float32)
    h = jnp.maximum(t + b1_ref[...], 0.0)
    o_ref[...] = jnp.dot(h, w2_ref[...], preferred_element_type=jnp.float32)


def _layer2_kernel(adj_ref, s2_ref, b2_ref, o_ref):
    t = jnp.dot(adj_ref[...], s2_ref[...], preferred_element_type=jnp.float32)
    o_ref[...] = t + b2_ref[...]


def kernel(x, adj, W1, b1, W2, b2):
    n, _ = x.shape
    b1 = b1.reshape(1, -1)
    b2 = b2.reshape(1, -1)

    s1 = pl.pallas_call(
        _xw_kernel,
        out_shape=jax.ShapeDtypeStruct((n, W1.shape[1]), jnp.float32),
    )(x, W1)

    grid = (n // _BM,)
    row_spec = pl.BlockSpec((_BM, n), lambda i: (i, 0))
    out_spec = pl.BlockSpec((_BM, _F), lambda i: (i, 0))

    s2 = pl.pallas_call(
        _layer1_kernel,
        grid=grid,
        in_specs=[
            row_spec,
            pl.BlockSpec((n, _F), lambda i: (0, 0)),
            pl.BlockSpec((1, _F), lambda i: (0, 0)),
            pl.BlockSpec((_F, _F), lambda i: (0, 0)),
        ],
        out_specs=out_spec,
        out_shape=jax.ShapeDtypeStruct((n, _F), jnp.float32),
    )(adj, s1, b1, W2)

    out = pl.pallas_call(
        _layer2_kernel,
        grid=grid,
        in_specs=[
            row_spec,
            pl.BlockSpec((n, _F), lambda i: (0, 0)),
            pl.BlockSpec((1, _F), lambda i: (0, 0)),
        ],
        out_specs=out_spec,
        out_shape=jax.ShapeDtypeStruct((n, _F), jnp.float32),
    )(adj, s2, b2)

    return out
